# baseline (device time: 93170 ns/iter reference)
import jax
import jax.numpy as jnp
from jax import lax
from jax.experimental import pallas as pl
from jax.experimental.pallas import tpu as pltpu

N_DEV = 16
NT = 4
NZ = 4
B_LOC = 2
SQ = 128
SKV = 128
HQ_LOC = 4
DH = 64
D_MODEL = 512
HD_LOC = HQ_LOC * DH
BF16 = jnp.bfloat16


def _body(x_ref, wq_ref, k_ref, v_ref, wo_ref, out_ref,
          x_full, wqP, woP, q_buf, ctx_buf, acc, stage, rs_in,
          wq_send, wo_send, x_send, rs_send,
          wq_recv, wo_recv, x_below, x_above, rs_sem):
    my = lax.axis_index("i")
    t = lax.rem(my, NT)
    z = my // NT
    mL = NT * z + lax.rem(t + NT - 1, NT)
    mR = NT * z + lax.rem(t + 1, NT)
    up = my + NT
    dn = my - NT
    has_up = z < NZ - 1
    has_dn = z > 0

    x_full[z] = x_ref[...]
    wqP[t] = wq_ref[...]
    woP[t] = wo_ref[...]
    acc[...] = jnp.zeros_like(acc)
    rs_in[...] = jnp.zeros_like(rs_in)

    barrier_sem = pltpu.get_barrier_semaphore()
    for nbr in (mL, mR):
        pl.semaphore_signal(barrier_sem, inc=1, device_id=(nbr,),
                            device_id_type=pl.DeviceIdType.MESH)

    @pl.when(has_up)
    def _():
        pl.semaphore_signal(barrier_sem, inc=1, device_id=(up,),
                            device_id_type=pl.DeviceIdType.MESH)

    @pl.when(has_dn)
    def _():
        pl.semaphore_signal(barrier_sem, inc=1, device_id=(dn,),
                            device_id_type=pl.DeviceIdType.MESH)

    pl.semaphore_wait(barrier_sem, 2)

    @pl.when(has_up)
    def _():
        pl.semaphore_wait(barrier_sem, 1)

    @pl.when(has_dn)
    def _():
        pl.semaphore_wait(barrier_sem, 1)

    ri = lax.broadcasted_iota(jnp.int32, (SQ, SKV), 0)
    ci = lax.broadcasted_iota(jnp.int32, (SQ, SKV), 1)
    mask01 = (ri // 64 == ci // 64).astype(jnp.float32)

    pending_sends = []

    def compute(zi, tj):
        hbase = (NT * z + tj) * HQ_LOC
        q = jnp.dot(x_full[zi], wqP[tj], preferred_element_type=jnp.float32)
        q_buf[...] = q.astype(BF16)
        for b in range(B_LOC):
            for hh in range(HQ_LOC):
                h_glob = hbase + hh
                qbh = q_buf[b * SQ:(b + 1) * SQ, hh * DH:(hh + 1) * DH]
                s = lax.dot_general(
                    qbh, k_ref[zi, b, h_glob], (((1,), (1,)), ((), ())),
                    preferred_element_type=jnp.float32,
                )
                w = jnp.exp(s) * mask01
                c = jnp.dot(w.astype(BF16), v_ref[zi, b, h_glob],
                            preferred_element_type=jnp.float32)
                c = c * (1.0 / jnp.sum(w, axis=1, keepdims=True))
                ctx_buf[b * SQ:(b + 1) * SQ, hh * DH:(hh + 1) * DH] = (
                    c.astype(BF16))
        acc[zi] += jnp.dot(ctx_buf[...], woP[tj],
                           preferred_element_type=jnp.float32)

    def x_desc(st, dir_up):
        slot = z - st if dir_up else z + st
        return pltpu.make_async_remote_copy(
            src_ref=x_full.at[slot], dst_ref=x_full.at[slot],
            send_sem=x_send.at[2 * st + (0 if dir_up else 1)],
            recv_sem=(x_below if dir_up else x_above).at[st],
            device_id=(up if dir_up else dn,),
            device_id_type=pl.DeviceIdType.MESH,
        )

    def start_x(st):
        up_ok = jnp.logical_and(z - st >= 0, has_up)
        dn_ok = jnp.logical_and(z + st <= NZ - 1, has_dn)

        @pl.when(up_ok)
        def _():
            x_desc(st, True).start()

        @pl.when(dn_ok)
        def _():
            x_desc(st, False).start()

        pending_sends.append((lambda: x_desc(st, True), up_ok))
        pending_sends.append((lambda: x_desc(st, False), dn_ok))

    def send_partial(zi, cond):
        stage[zi] = acc[zi].astype(BF16)
        d = pltpu.make_async_remote_copy(
            src_ref=stage.at[zi], dst_ref=rs_in.at[z],
            send_sem=rs_send.at[zi], recv_sem=rs_sem.at[z],
            device_id=(NT * zi + t,), device_id_type=pl.DeviceIdType.MESH,
        )
        d.start()
        pending_sends.append((
            lambda: pltpu.make_async_remote_copy(
                src_ref=stage.at[zi], dst_ref=rs_in.at[z],
                send_sem=rs_send.at[zi], recv_sem=rs_sem.at[z],
                device_id=(NT * zi + t,),
                device_id_type=pl.DeviceIdType.MESH),
            cond))

    start_x(0)
    w_hop = []
    for (full, send_sem, recv_sem) in ((wqP, wq_send, wq_recv),
                                       (woP, wo_send, wo_recv)):
        w_hop.append(pltpu.make_async_remote_copy(
            src_ref=full.at[t], dst_ref=full.at[t],
            send_sem=send_sem.at[0], recv_sem=recv_sem.at[0],
            device_id=(mR,), device_id_type=pl.DeviceIdType.MESH))
        w_hop.append(pltpu.make_async_remote_copy(
            src_ref=full.at[t], dst_ref=full.at[t],
            send_sem=send_sem.at[1], recv_sem=recv_sem.at[2],
            device_id=(mL,), device_id_type=pl.DeviceIdType.MESH))
    for d in w_hop:
        d.start()

    compute(z, t)

    for d in w_hop:
        d.wait()

    tm1 = lax.rem(t + NT - 1, NT)
    w_hop1 = []
    for (full, send_sem, recv_sem) in ((wqP, wq_send, wq_recv),
                                       (woP, wo_send, wo_recv)):
        w_hop1.append(pltpu.make_async_remote_copy(
            src_ref=full.at[tm1], dst_ref=full.at[tm1],
            send_sem=send_sem.at[0], recv_sem=recv_sem.at[1],
            device_id=(mR,), device_id_type=pl.DeviceIdType.MESH))
    for d in w_hop1:
        d.start()

    compute(z, lax.rem(t + 1, NT))
    compute(z, tm1)

    for d in w_hop1:
        d.wait()

    compute(z, lax.rem(t + 2, NT))

    for st in range(NZ - 1):
        below_ok = z >= st + 1
        above_ok = z <= NZ - 2 - st
        zb = jnp.maximum(z - 1 - st, 0)
        za = jnp.minimum(z + 1 + st, NZ - 1)

        @pl.when(below_ok)
        def _(st=st):
            pltpu.make_async_remote_copy(
                src_ref=x_full.at[z - 1 - st], dst_ref=x_full.at[z - 1 - st],
                send_sem=x_send.at[0], recv_sem=x_below.at[st],
                device_id=(dn,), device_id_type=pl.DeviceIdType.MESH,
            ).wait_recv()

        @pl.when(above_ok)
        def _(st=st):
            pltpu.make_async_remote_copy(
                src_ref=x_full.at[z + 1 + st], dst_ref=x_full.at[z + 1 + st],
                send_sem=x_send.at[0], recv_sem=x_above.at[st],
                device_id=(up,), device_id_type=pl.DeviceIdType.MESH,
            ).wait_recv()

        if st < NZ - 2:
            start_x(st + 1)

        @pl.when(below_ok)
        def _(zb=zb, below_ok=below_ok):
            for tj in range(NT):
                compute(zb, tj)
            send_partial(zb, below_ok)

        @pl.when(above_ok)
        def _(za=za, above_ok=above_ok):
            for tj in range(NT):
                compute(za, tj)
            send_partial(za, above_ok)

    for zz in range(NZ):
        @pl.when(zz != z)
        def _(zz=zz):
            pltpu.make_async_remote_copy(
                src_ref=stage.at[0], dst_ref=rs_in.at[zz],
                send_sem=rs_send.at[0], recv_sem=rs_sem.at[zz],
                device_id=(my,), device_id_type=pl.DeviceIdType.MESH,
            ).wait_recv()

    total = acc[z]
    for zz in range(NZ):
        total += rs_in[zz].astype(jnp.float32)
    out_ref[...] = total

    for mk, cond in pending_sends:
        @pl.when(cond)
        def _(mk=mk):
            mk().wait_send()


def kernel(x, Wq, K_ext, V_ext, Wo):
    my = lax.axis_index("i")
    t = lax.rem(my, NT)
    x2 = (x.reshape(B_LOC * SQ, D_MODEL) * 0.125).astype(BF16)
    wq = Wq.astype(BF16)
    wo = Wo.astype(BF16)
    K6 = K_ext.reshape(NZ, NT, B_LOC, SKV, HQ_LOC * N_DEV, DH)
    V6 = V_ext.reshape(NZ, NT, B_LOC, SKV, HQ_LOC * N_DEV, DH)
    k_c = jnp.transpose(K6[:, t], (0, 1, 3, 2, 4)).astype(BF16)
    v_c = jnp.transpose(V6[:, t], (0, 1, 3, 2, 4)).astype(BF16)

    out2 = pl.pallas_call(
        _body,
        out_shape=jax.ShapeDtypeStruct((B_LOC * SQ, D_MODEL), jnp.float32),
        in_specs=[pl.BlockSpec(memory_space=pltpu.VMEM)] * 5,
        out_specs=pl.BlockSpec(memory_space=pltpu.VMEM),
        scratch_shapes=[
            pltpu.VMEM((NZ, B_LOC * SQ, D_MODEL), BF16),
            pltpu.VMEM((NT, D_MODEL, HD_LOC), BF16),
            pltpu.VMEM((NT, HD_LOC, D_MODEL), BF16),
            pltpu.VMEM((B_LOC * SQ, HD_LOC), BF16),
            pltpu.VMEM((B_LOC * SQ, HD_LOC), BF16),
            pltpu.VMEM((NZ, B_LOC * SQ, D_MODEL), jnp.float32),
            pltpu.VMEM((NZ, B_LOC * SQ, D_MODEL), BF16),
            pltpu.VMEM((NZ, B_LOC * SQ, D_MODEL), BF16),
            pltpu.SemaphoreType.DMA((2,)),
            pltpu.SemaphoreType.DMA((2,)),
            pltpu.SemaphoreType.DMA((6,)),
            pltpu.SemaphoreType.DMA((NZ,)),
            pltpu.SemaphoreType.DMA((3,)),
            pltpu.SemaphoreType.DMA((3,)),
            pltpu.SemaphoreType.DMA((3,)),
            pltpu.SemaphoreType.DMA((3,)),
            pltpu.SemaphoreType.DMA((NZ,)),
        ],
        compiler_params=pltpu.CompilerParams(collective_id=0),
    )(x2, wq, k_c, v_c, wo)
    return out2.reshape(B_LOC, SQ, D_MODEL)


# device time: 74430 ns/iter; 1.2518x vs baseline; 1.2518x over previous
import jax
import jax.numpy as jnp
from jax import lax
from jax.experimental import pallas as pl
from jax.experimental.pallas import tpu as pltpu

N_DEV = 16
B_LOC = 2
SQ = 128
SKV = 128
HQ_LOC = 4
DH = 64
D_MODEL = 512
HD_LOC = HQ_LOC * DH
BF16 = jnp.bfloat16


_PI = (0, 1, 5, 9, 13, 14, 10, 6, 2, 3, 7, 11, 15, 12, 8, 4)
_INV_PI = (0, 1, 8, 9, 15, 2, 7, 10, 14, 3, 6, 11, 13, 4, 5, 12)


def _lookup(table, q):
    o = jnp.int32(0)
    for p in range(N_DEV):
        o = jnp.where(q == p, jnp.int32(table[p]), o)
    return o


def _body(x_ref, wq_ref, k_ref, v_ref, wo_ref, out_ref,
          wq_full, wo_full, q_buf, ctx_buf, acc,
          wq_send, wo_send, wq_recv, wo_recv):
    my = lax.axis_index("i")
    p_my = _lookup(_INV_PI, my)
    left = _lookup(_PI, lax.rem(p_my + N_DEV - 1, N_DEV))
    right = _lookup(_PI, lax.rem(p_my + 1, N_DEV))

    wq_full[p_my] = wq_ref[...]
    wo_full[p_my] = wo_ref[...]

    barrier_sem = pltpu.get_barrier_semaphore()
    for nbr in (left, right):
        pl.semaphore_signal(
            barrier_sem, inc=1,
            device_id=(nbr,), device_id_type=pl.DeviceIdType.MESH,
        )
    pl.semaphore_wait(barrier_sem, 2)

    acc[...] = jnp.zeros_like(acc)

    ri = lax.broadcasted_iota(jnp.int32, (SQ, SKV), 0)
    ci = lax.broadcasted_iota(jnp.int32, (SQ, SKV), 1)
    mask01 = (ri // 64 == ci // 64).astype(jnp.float32)

    def compute(slot):
        o = _lookup(_PI, slot)
        q = jnp.dot(x_ref[...], wq_full[slot],
                    preferred_element_type=jnp.float32)
        q_buf[...] = q.astype(BF16)
        for b in range(B_LOC):
            for hh in range(HQ_LOC):
                h_glob = o * HQ_LOC + hh
                qbh = q_buf[b * SQ:(b + 1) * SQ, hh * DH:(hh + 1) * DH]
                kbh = k_ref[b, h_glob]
                s = lax.dot_general(
                    qbh, kbh, (((1,), (1,)), ((), ())),
                    preferred_element_type=jnp.float32,
                )
                w = jnp.exp(s) * mask01
                c = jnp.dot(w.astype(BF16), v_ref[b, h_glob],
                            preferred_element_type=jnp.float32)
                c = c * (1.0 / jnp.sum(w, axis=1, keepdims=True))
                ctx_buf[b * SQ:(b + 1) * SQ, hh * DH:(hh + 1) * DH] = (
                    c.astype(BF16))
        acc[...] += jnp.dot(ctx_buf[...], wo_full[slot],
                            preferred_element_type=jnp.float32)

    R_HOPS = N_DEV // 2
    L_HOPS = N_DEV - 1 - R_HOPS
    for h in range(R_HOPS):
        r_slot = lax.rem(p_my + 2 * N_DEV - h, N_DEV)
        l_slot = lax.rem(p_my + h, N_DEV)
        rdmas = []
        for (full, send_sem, recv_sem) in (
            (wq_full, wq_send, wq_recv),
            (wo_full, wo_send, wo_recv),
        ):
            rdmas.append(pltpu.make_async_remote_copy(
                src_ref=full.at[r_slot], dst_ref=full.at[r_slot],
                send_sem=send_sem.at[0], recv_sem=recv_sem.at[h],
                device_id=(right,), device_id_type=pl.DeviceIdType.MESH,
            ))
            if h < L_HOPS:
                rdmas.append(pltpu.make_async_remote_copy(
                    src_ref=full.at[l_slot], dst_ref=full.at[l_slot],
                    send_sem=send_sem.at[1],
                    recv_sem=recv_sem.at[R_HOPS + h],
                    device_id=(left,), device_id_type=pl.DeviceIdType.MESH,
                ))
        for r in rdmas:
            r.start()
        if h == 0:
            compute(p_my)
        else:
            compute(r_slot)
            compute(l_slot)
        for r in rdmas:
            r.wait()
    compute(lax.rem(p_my + N_DEV // 2, N_DEV))

    out_ref[...] = acc[...]


def kernel(x, Wq, K_ext, V_ext, Wo):
    my = lax.axis_index("i")
    x2 = (x.reshape(B_LOC * SQ, D_MODEL) * 0.125).astype(BF16)
    wq = Wq.astype(BF16)
    wo = Wo.astype(BF16)
    k_loc = lax.dynamic_slice_in_dim(K_ext, my * B_LOC, B_LOC, axis=0)
    v_loc = lax.dynamic_slice_in_dim(V_ext, my * B_LOC, B_LOC, axis=0)
    k_loc = jnp.transpose(k_loc, (0, 2, 1, 3)).astype(BF16)
    v_loc = jnp.transpose(v_loc, (0, 2, 1, 3)).astype(BF16)

    out2 = pl.pallas_call(
        _body,
        out_shape=jax.ShapeDtypeStruct((B_LOC * SQ, D_MODEL), jnp.float32),
        in_specs=[pl.BlockSpec(memory_space=pltpu.VMEM)] * 5,
        out_specs=pl.BlockSpec(memory_space=pltpu.VMEM),
        scratch_shapes=[
            pltpu.VMEM((N_DEV, D_MODEL, HD_LOC), BF16),
            pltpu.VMEM((N_DEV, HD_LOC, D_MODEL), BF16),
            pltpu.VMEM((B_LOC * SQ, HD_LOC), BF16),
            pltpu.VMEM((B_LOC * SQ, HD_LOC), BF16),
            pltpu.VMEM((B_LOC * SQ, D_MODEL), jnp.float32),
            pltpu.SemaphoreType.DMA((2,)),
            pltpu.SemaphoreType.DMA((2,)),
            pltpu.SemaphoreType.DMA((N_DEV - 1,)),
            pltpu.SemaphoreType.DMA((N_DEV - 1,)),
        ],
        compiler_params=pltpu.CompilerParams(collective_id=0),
    )(x2, wq, k_loc, v_loc, wo)
    return out2.reshape(B_LOC, SQ, D_MODEL)
